# baseline (speedup 1.0000x reference)
import functools

import jax
import jax.numpy as jnp
from jax import lax
from jax.experimental import pallas as pl
from jax.experimental.pallas import tpu as pltpu
from jax.experimental.pallas import tpu_sc as plsc

V_TILE = 2048


def _gather_sc(emb_table, input_ids):
    info = plsc.get_sparse_core_info()
    nc, ns = info.num_cores, info.num_subcores
    nw = nc * ns
    b = input_ids.shape[0]
    d = emb_table.shape[1]
    b_per_w = b // nw
    mesh = plsc.VectorSubcoreMesh(core_axis_name="c", subcore_axis_name="s")

    @functools.partial(
        pl.kernel,
        mesh=mesh,
        out_type=jax.ShapeDtypeStruct((b, d), jnp.float32),
        scratch_types=[
            pltpu.VMEM((b_per_w,), jnp.int32),
            pltpu.VMEM((b_per_w, d), jnp.float32),
            pltpu.SemaphoreType.DMA,
        ],
        compiler_params=pltpu.CompilerParams(use_tc_tiling_on_sc=False),
    )
    def gather_kernel(table_hbm, idx_hbm, out_hbm, idx_v, rows_v, sem):
        wid = lax.axis_index("s") * nc + lax.axis_index("c")
        base = wid * b_per_w
        pltpu.sync_copy(idx_hbm.at[pl.ds(base, b_per_w)], idx_v)
        pltpu.async_copy(table_hbm.at[idx_v], rows_v, sem).wait()
        pltpu.sync_copy(rows_v, out_hbm.at[pl.ds(base, b_per_w)])

    return gather_kernel(emb_table, input_ids)


def _project_body(h_ref, w_ref, b_ref, out_ref):
    out_ref[...] = (
        lax.dot_general(
            h_ref[...],
            w_ref[...],
            dimension_numbers=(((1,), (1,)), ((), ())),
            preferred_element_type=jnp.float32,
        )
        + b_ref[...]
    )


def _project_tc(h, head_w, head_b):
    b, hid = h.shape
    v = head_w.shape[0]
    return pl.pallas_call(
        _project_body,
        grid=(pl.cdiv(v, V_TILE),),
        in_specs=[
            pl.BlockSpec((b, hid), lambda j: (0, 0)),
            pl.BlockSpec((V_TILE, hid), lambda j: (j, 0)),
            pl.BlockSpec((1, V_TILE), lambda j: (0, j)),
        ],
        out_specs=pl.BlockSpec((b, V_TILE), lambda j: (0, j)),
        out_shape=jax.ShapeDtypeStruct((b, v), jnp.float32),
        compiler_params=pltpu.CompilerParams(
            dimension_semantics=("arbitrary",)
        ),
    )(h, head_w, head_b.reshape(1, v))


def kernel(input_ids, emb_table, head_w, head_b):
    h = _gather_sc(emb_table, input_ids)
    return _project_tc(h, head_w, head_b)


# speedup vs baseline: 1.0773x; 1.0773x over previous
import functools

import jax
import jax.numpy as jnp
from jax import lax
from jax.experimental import pallas as pl
from jax.experimental.pallas import tpu as pltpu
from jax.experimental.pallas import tpu_sc as plsc

V_TILE = 2048


def _gather_sc(emb_table, input_ids):
    info = plsc.get_sparse_core_info()
    nc, ns = info.num_cores, info.num_subcores
    nw = nc * ns
    b = input_ids.shape[0]
    d = emb_table.shape[1]
    b_per_w = b // nw
    mesh = plsc.VectorSubcoreMesh(core_axis_name="c", subcore_axis_name="s")

    @functools.partial(
        pl.kernel,
        mesh=mesh,
        out_type=jax.ShapeDtypeStruct((b, d), jnp.float32),
        scratch_types=[
            pltpu.VMEM((b_per_w,), jnp.int32),
            pltpu.VMEM((b_per_w, d), jnp.float32),
            pltpu.SemaphoreType.DMA,
        ],
        compiler_params=pltpu.CompilerParams(use_tc_tiling_on_sc=False),
    )
    def gather_kernel(table_hbm, idx_hbm, out_hbm, idx_v, rows_v, sem):
        wid = lax.axis_index("s") * nc + lax.axis_index("c")
        base = wid * b_per_w
        pltpu.sync_copy(idx_hbm.at[pl.ds(base, b_per_w)], idx_v)
        pltpu.async_copy(table_hbm.at[idx_v], rows_v, sem).wait()
        pltpu.sync_copy(rows_v, out_hbm.at[pl.ds(base, b_per_w)])

    return gather_kernel(emb_table, input_ids)


def _project_body(h_ref, wt_ref, b_ref, out_ref):
    out_ref[...] = (
        lax.dot_general(
            h_ref[...],
            wt_ref[...],
            dimension_numbers=(((1,), (0,)), ((), ())),
            preferred_element_type=jnp.float32,
        )
        + b_ref[...]
    )


def _project_tc(h, head_wt, head_b):
    b, hid = h.shape
    v = head_wt.shape[1]
    return pl.pallas_call(
        _project_body,
        grid=(pl.cdiv(v, V_TILE),),
        in_specs=[
            pl.BlockSpec((b, hid), lambda j: (0, 0)),
            pl.BlockSpec((hid, V_TILE), lambda j: (0, j)),
            pl.BlockSpec((1, V_TILE), lambda j: (0, j)),
        ],
        out_specs=pl.BlockSpec((b, V_TILE), lambda j: (0, j)),
        out_shape=jax.ShapeDtypeStruct((b, v), jnp.float32),
        compiler_params=pltpu.CompilerParams(
            dimension_semantics=("arbitrary",),
        ),
    )(h, head_wt, head_b.reshape(1, v))


def kernel(input_ids, emb_table, head_w, head_b):
    h = _gather_sc(emb_table, input_ids)
    return _project_tc(h, head_w.T, head_b)


# speedup vs baseline: 1.0780x; 1.0007x over previous
import functools

import jax
import jax.numpy as jnp
from jax import lax
from jax.experimental import pallas as pl
from jax.experimental.pallas import tpu as pltpu
from jax.experimental.pallas import tpu_sc as plsc

V_TILE = 4096


def _gather_sc(emb_table, input_ids):
    info = plsc.get_sparse_core_info()
    nc, ns = info.num_cores, info.num_subcores
    nw = nc * ns
    b = input_ids.shape[0]
    d = emb_table.shape[1]
    b_per_w = b // nw
    mesh = plsc.VectorSubcoreMesh(core_axis_name="c", subcore_axis_name="s")

    @functools.partial(
        pl.kernel,
        mesh=mesh,
        out_type=jax.ShapeDtypeStruct((b, d), jnp.float32),
        scratch_types=[
            pltpu.VMEM((b_per_w,), jnp.int32),
            pltpu.VMEM((b_per_w, d), jnp.float32),
            pltpu.SemaphoreType.DMA,
        ],
        compiler_params=pltpu.CompilerParams(use_tc_tiling_on_sc=False),
    )
    def gather_kernel(table_hbm, idx_hbm, out_hbm, idx_v, rows_v, sem):
        wid = lax.axis_index("s") * nc + lax.axis_index("c")
        base = wid * b_per_w
        pltpu.sync_copy(idx_hbm.at[pl.ds(base, b_per_w)], idx_v)
        pltpu.async_copy(table_hbm.at[idx_v], rows_v, sem).wait()
        pltpu.sync_copy(rows_v, out_hbm.at[pl.ds(base, b_per_w)])

    return gather_kernel(emb_table, input_ids)


def _project_body(h_ref, wt_ref, b_ref, out_ref):
    out_ref[...] = (
        lax.dot_general(
            h_ref[...],
            wt_ref[...],
            dimension_numbers=(((1,), (0,)), ((), ())),
            preferred_element_type=jnp.float32,
        )
        + b_ref[...]
    )


def _project_tc(h, head_wt, head_b):
    b, hid = h.shape
    v = head_wt.shape[1]
    return pl.pallas_call(
        _project_body,
        grid=(pl.cdiv(v, V_TILE),),
        in_specs=[
            pl.BlockSpec((b, hid), lambda j: (0, 0)),
            pl.BlockSpec((hid, V_TILE), lambda j: (0, j)),
            pl.BlockSpec((1, V_TILE), lambda j: (0, j)),
        ],
        out_specs=pl.BlockSpec((b, V_TILE), lambda j: (0, j)),
        out_shape=jax.ShapeDtypeStruct((b, v), jnp.float32),
        compiler_params=pltpu.CompilerParams(
            dimension_semantics=("arbitrary",),
        ),
    )(h, head_wt, head_b.reshape(1, v))


def kernel(input_ids, emb_table, head_w, head_b):
    h = _gather_sc(emb_table, input_ids)
    return _project_tc(h, head_w.T, head_b)


# speedup vs baseline: 1.0875x; 1.0088x over previous
import functools

import jax
import jax.numpy as jnp
from jax import lax
from jax.experimental import pallas as pl
from jax.experimental.pallas import tpu as pltpu
from jax.experimental.pallas import tpu_sc as plsc

V_TILE = 4096
LANES = 128


def _gather_sc(emb_pad, input_ids):
    info = plsc.get_sparse_core_info()
    nc, ns = info.num_cores, info.num_subcores
    nw = nc * ns
    b = input_ids.shape[0]
    d = emb_pad.shape[1]
    b_per_w = b // nw
    mesh = plsc.VectorSubcoreMesh(core_axis_name="c", subcore_axis_name="s")

    @functools.partial(
        pl.kernel,
        mesh=mesh,
        out_type=jax.ShapeDtypeStruct((b, d), jnp.float32),
        scratch_types=[
            pltpu.VMEM((b_per_w,), jnp.int32),
            pltpu.VMEM((b_per_w, d), jnp.float32),
            pltpu.SemaphoreType.DMA,
        ],
    )
    def gather_kernel(table_hbm, idx_hbm, out_hbm, idx_v, rows_v, sem):
        wid = lax.axis_index("s") * nc + lax.axis_index("c")
        base = wid * b_per_w
        pltpu.sync_copy(idx_hbm.at[pl.ds(base, b_per_w)], idx_v)
        pltpu.async_copy(table_hbm.at[idx_v], rows_v, sem).wait()
        pltpu.sync_copy(rows_v, out_hbm.at[pl.ds(base, b_per_w)])

    return gather_kernel(emb_pad, input_ids)


def _project_body(h_ref, wt_ref, b_ref, out_ref):
    h = h_ref[:, 0:32]
    out_ref[...] = (
        lax.dot_general(
            h,
            wt_ref[...],
            dimension_numbers=(((1,), (0,)), ((), ())),
            preferred_element_type=jnp.float32,
        )
        + b_ref[...]
    )


def _project_tc(h_pad, head_wt, head_b):
    b = h_pad.shape[0]
    hid, v = head_wt.shape
    return pl.pallas_call(
        _project_body,
        grid=(pl.cdiv(v, V_TILE),),
        in_specs=[
            pl.BlockSpec((b, LANES), lambda j: (0, 0)),
            pl.BlockSpec((hid, V_TILE), lambda j: (0, j)),
            pl.BlockSpec((1, V_TILE), lambda j: (0, j)),
        ],
        out_specs=pl.BlockSpec((b, V_TILE), lambda j: (0, j)),
        out_shape=jax.ShapeDtypeStruct((b, v), jnp.float32),
        compiler_params=pltpu.CompilerParams(
            dimension_semantics=("arbitrary",),
        ),
    )(h_pad, head_wt, head_b.reshape(1, v))


def kernel(input_ids, emb_table, head_w, head_b):
    emb_pad = jnp.pad(emb_table, ((0, 0), (0, LANES - emb_table.shape[1])))
    h_pad = _gather_sc(emb_pad, input_ids)
    return _project_tc(h_pad, head_w.T, head_b)


# speedup vs baseline: 3.0503x; 2.8049x over previous
import functools

import jax
import jax.numpy as jnp
from jax import lax
from jax.experimental import pallas as pl
from jax.experimental.pallas import tpu as pltpu
from jax.experimental.pallas import tpu_sc as plsc

V_TILE = 4096
LANES = 128


def _gather_sc(emb_pad, input_ids):
    info = plsc.get_sparse_core_info()
    nc, ns = info.num_cores, info.num_subcores
    nw = nc * ns
    b = input_ids.shape[0]
    d = emb_pad.shape[1]
    b_per_w = b // nw
    mesh = plsc.VectorSubcoreMesh(core_axis_name="c", subcore_axis_name="s")

    @functools.partial(
        pl.kernel,
        mesh=mesh,
        out_type=jax.ShapeDtypeStruct((b, d), jnp.float32),
        scratch_types=[
            pltpu.VMEM((b_per_w,), jnp.int32),
            pltpu.VMEM((b_per_w, d), jnp.float32),
            pltpu.SemaphoreType.DMA,
        ],
    )
    def gather_kernel(table_hbm, idx_hbm, out_hbm, idx_v, rows_v, sem):
        wid = lax.axis_index("s") * nc + lax.axis_index("c")
        base = wid * b_per_w
        pltpu.sync_copy(idx_hbm.at[pl.ds(base, b_per_w)], idx_v)
        pltpu.async_copy(table_hbm.at[idx_v], rows_v, sem).wait()
        pltpu.sync_copy(rows_v, out_hbm.at[pl.ds(base, b_per_w)])

    return gather_kernel(emb_pad, input_ids)


def _project_body(h_ref, wt_ref, b_ref, out_ref):
    h33 = h_ref[:, 0:33]
    ones_col = (
        lax.broadcasted_iota(jnp.int32, h33.shape, dimension=1) == 32
    ).astype(jnp.float32)
    h_aug = h33 + ones_col
    w_aug = jnp.concatenate([wt_ref[...], b_ref[...]], axis=0)
    out_ref[...] = lax.dot_general(
        w_aug,
        h_aug,
        dimension_numbers=(((0,), (1,)), ((), ())),
        preferred_element_type=jnp.float32,
    )


def _project_tc(h_pad, head_wt, head_b):
    b = h_pad.shape[0]
    hid, v = head_wt.shape
    out_t = pl.pallas_call(
        _project_body,
        grid=(pl.cdiv(v, V_TILE),),
        in_specs=[
            pl.BlockSpec((b, LANES), lambda j: (0, 0)),
            pl.BlockSpec((hid, V_TILE), lambda j: (0, j)),
            pl.BlockSpec((1, V_TILE), lambda j: (0, j)),
        ],
        out_specs=pl.BlockSpec((V_TILE, b), lambda j: (j, 0)),
        out_shape=jax.ShapeDtypeStruct((v, b), jnp.float32),
        compiler_params=pltpu.CompilerParams(
            dimension_semantics=("arbitrary",),
        ),
    )(h_pad, head_wt, head_b.reshape(1, v))
    return out_t.T


def kernel(input_ids, emb_table, head_w, head_b):
    emb_pad = jnp.pad(emb_table, ((0, 0), (0, LANES - emb_table.shape[1])))
    h_pad = _gather_sc(emb_pad, input_ids)
    return _project_tc(h_pad, head_w.T, head_b)
